# Initial kernel scaffold; baseline (speedup 1.0000x reference)
import jax
import jax.numpy as jnp
from jax.experimental import pallas as pl


def kernel(x, Wr, br, W1, b1, W2, b2):
    raise NotImplementedError("write your pallas kernel here")


# speedup vs baseline: 1.8211x; 1.8211x over previous
import jax
import jax.numpy as jnp
from jax import lax
from jax.experimental import pallas as pl
from jax.experimental.pallas import tpu as pltpu
from jax.experimental.pallas import tpu_sc as plsc

N_EXP = 8
TOPK = 2
T = 2048
D = 768
H = 2 * D
BLK = 256
NB = 24
P = NB * BLK
NW = 32
TW = T // NW
CHUNK = 128
NSLOT = 2
NSPLIT = 1


def _router_body(x_ref, wr_ref, br_ref,
                 pos0_ref, pos1_ref, g0_ref, g1_ref, bexp_ref, nact_ref,
                 wslot_ref, wchg_ref, prefe_ref, dopref_ref, pslot_ref,
                 ntrans_ref, trans_ref):
    x = x_ref[...]
    logits = jnp.dot(x, wr_ref[...],
                     preferred_element_type=jnp.float32) + br_ref[...]
    iota_e = lax.broadcasted_iota(jnp.int32, (T, N_EXP), 1)

    m1 = jnp.max(logits, axis=1, keepdims=True)
    idx1 = jnp.min(jnp.where(logits == m1, iota_e, N_EXP), axis=1,
                   keepdims=True)
    sel1 = iota_e == idx1
    masked = jnp.where(sel1, -jnp.inf, logits)
    m2 = jnp.max(masked, axis=1, keepdims=True)
    idx2 = jnp.min(jnp.where(masked == m2, iota_e, N_EXP), axis=1,
                   keepdims=True)
    sel2 = iota_e == idx2

    e2 = jnp.exp(m2 - m1)
    den = 1.0 + e2
    g0 = 1.0 / den
    g1 = e2 / den

    oh = jnp.concatenate([sel1.astype(jnp.float32),
                          sel2.astype(jnp.float32)], axis=1)

    tri = (lax.broadcasted_iota(jnp.int32, (CHUNK, CHUNK), 0) >=
           lax.broadcasted_iota(jnp.int32, (CHUNK, CHUNK), 1)
           ).astype(jnp.float32)
    csum_parts = []
    carry = jnp.zeros((1, 2 * N_EXP), jnp.float32)
    for g in range(T // CHUNK):
        blk = oh[g * CHUNK:(g + 1) * CHUNK, :]
        cg = jnp.dot(tri, blk, preferred_element_type=jnp.float32) + carry
        carry = cg[CHUNK - 1:CHUNK, :]
        csum_parts.append(cg)
    csum = jnp.concatenate(csum_parts, axis=0)
    c0 = csum[:, :N_EXP]
    c1 = csum[:, N_EXP:]
    tot0 = carry[:, :N_EXP]
    tot1 = carry[:, N_EXP:]
    counts = tot0 + tot1

    nblk = jnp.ceil(counts * (1.0 / BLK))
    sut = (lax.broadcasted_iota(jnp.int32, (N_EXP, N_EXP), 0) <
           lax.broadcasted_iota(jnp.int32, (N_EXP, N_EXP), 1)
           ).astype(jnp.float32)
    bstart = jnp.dot(nblk, sut, preferred_element_type=jnp.float32)
    boff = bstart * float(BLK)

    base0 = jnp.sum(jnp.where(sel1, boff, 0.0), axis=1, keepdims=True)
    rank0 = jnp.sum(jnp.where(sel1, c0, 0.0), axis=1, keepdims=True) - 1.0
    base1 = jnp.sum(jnp.where(sel2, boff + tot0, 0.0), axis=1, keepdims=True)
    rank1 = jnp.sum(jnp.where(sel2, c1, 0.0), axis=1, keepdims=True) - 1.0
    pos0_ref[...] = (base0 + rank0).astype(jnp.int32)
    pos1_ref[...] = (base1 + rank1).astype(jnp.int32)

    g0_ref[...] = jnp.broadcast_to(g0, (T, 16))
    g1_ref[...] = jnp.broadcast_to(g1, (T, 16))

    bend = bstart + nblk
    iota_b = lax.broadcasted_iota(jnp.int32, (NB, N_EXP), 0).astype(jnp.float32)
    bexp = jnp.sum((bend <= iota_b + 0.5).astype(jnp.int32), axis=1,
                   keepdims=True)
    bexp_ref[...] = jnp.minimum(bexp, N_EXP - 1)
    nact_ref[...] = jnp.sum(nblk, axis=1, keepdims=True).astype(jnp.int32)

    active = nblk > 0.5
    iota_e_f = lax.broadcasted_iota(jnp.int32, (NB, N_EXP), 1).astype(jnp.float32)
    starts_here = active & (jnp.abs(bstart - iota_b) < 0.5)
    wchg_ref[...] = jnp.max(starts_here.astype(jnp.int32), axis=1,
                            keepdims=True)
    nstarted = jnp.sum((active & (bstart <= iota_b + 0.5)).astype(jnp.int32),
                       axis=1, keepdims=True)
    tidx = jnp.maximum(nstarted - 1, 0)
    wslot_ref[...] = tidx % NSLOT

    active_f = active.astype(jnp.float32)
    rank = jnp.dot(active_f, sut,
                   preferred_element_type=jnp.float32)
    ntrans = jnp.sum(active_f, axis=1, keepdims=True)
    ntrans_ref[...] = ntrans.astype(jnp.int32)
    iota_s8 = lax.broadcasted_iota(jnp.int32, (N_EXP, N_EXP), 0).astype(jnp.float32)
    iota_c8 = lax.broadcasted_iota(jnp.int32, (N_EXP, N_EXP), 1).astype(jnp.float32)
    smask = active & (jnp.abs(rank - iota_s8) < 0.5)
    trans_ref[...] = jnp.sum(jnp.where(smask, iota_c8, 0.0), axis=1,
                             keepdims=True).astype(jnp.int32)

    tt = tidx.astype(jnp.float32) + float(NSLOT - 1)
    pmask = active & (jnp.abs(rank - tt) < 0.5)
    prefe_ref[...] = jnp.sum(jnp.where(pmask, iota_e_f, 0.0), axis=1,
                             keepdims=True).astype(jnp.int32)
    dopref_ref[...] = ((tt < ntrans - 0.5) & (tidx >= 1) &
                       (wchg_ref[...] == 1)).astype(jnp.int32)
    pslot_ref[...] = tt.astype(jnp.int32) % NSLOT


def _router_plan(x2, Wr, br2):
    return pl.pallas_call(
        _router_body,
        out_shape=(
            jax.ShapeDtypeStruct((T, 1), jnp.int32),
            jax.ShapeDtypeStruct((T, 1), jnp.int32),
            jax.ShapeDtypeStruct((T, 16), jnp.float32),
            jax.ShapeDtypeStruct((T, 16), jnp.float32),
            jax.ShapeDtypeStruct((NB, 1), jnp.int32),
            jax.ShapeDtypeStruct((1, 1), jnp.int32),
            jax.ShapeDtypeStruct((NB, 1), jnp.int32),
            jax.ShapeDtypeStruct((NB, 1), jnp.int32),
            jax.ShapeDtypeStruct((NB, 1), jnp.int32),
            jax.ShapeDtypeStruct((NB, 1), jnp.int32),
            jax.ShapeDtypeStruct((NB, 1), jnp.int32),
            jax.ShapeDtypeStruct((1, 1), jnp.int32),
            jax.ShapeDtypeStruct((N_EXP, 1), jnp.int32),
        ),
    )(x2, Wr, br2)


def _dispatch_body(x_hbm, pos0_hbm, pos1_hbm, xs_hbm,
                   idx0_v, idx1_v, rows_v, sem0, sem1):
    wid = lax.axis_index("s") * 2 + lax.axis_index("c")
    base = wid * TW
    ld0 = pltpu.async_copy(pos0_hbm.at[pl.ds(base, TW)], idx0_v, sem0)
    ld1 = pltpu.async_copy(pos1_hbm.at[pl.ds(base, TW)], idx1_v, sem1)
    pltpu.sync_copy(x_hbm.at[pl.ds(base, TW), :], rows_v)
    ld0.wait()
    ld1.wait()
    cp0 = pltpu.async_copy(rows_v, xs_hbm.at[idx0_v], sem0)
    cp1 = pltpu.async_copy(rows_v, xs_hbm.at[idx1_v], sem1)
    cp0.wait()
    cp1.wait()


def _dispatch(x2, pos0, pos1):
    mesh = plsc.VectorSubcoreMesh(core_axis_name="c", subcore_axis_name="s")
    return pl.kernel(
        _dispatch_body,
        out_type=jax.ShapeDtypeStruct((P, D), jnp.float32),
        mesh=mesh,
        scratch_types=[
            pltpu.VMEM((TW,), jnp.int32),
            pltpu.VMEM((TW,), jnp.int32),
            pltpu.VMEM((TW, D), jnp.float32),
            pltpu.SemaphoreType.DMA,
            pltpu.SemaphoreType.DMA,
        ],
    )(x2, pos0, pos1)


def _ffn_body(bexp_ref, nact_ref, wslot_ref, wchg_ref, prefe_ref, dopref_ref,
              pslot_ref, ntrans_ref, trans_ref,
              xs_ref, w1_hbm, b1_ref, w2_hbm, b2_ref, os_ref,
              w1_buf, w2_buf, sems):
    b = pl.program_id(0)
    e = bexp_ref[b]
    slot = wslot_ref[b]
    nact = nact_ref[0]

    def _w_copies(te, s):
        cps = []
        for k in range(NSPLIT):
            r1 = pl.ds(k * (D // NSPLIT), D // NSPLIT)
            cps.append(pltpu.make_async_copy(
                w1_hbm.at[te, r1, :], w1_buf.at[s, r1, :], sems.at[s, 0, k]))
            r2 = pl.ds(k * (H // NSPLIT), H // NSPLIT)
            cps.append(pltpu.make_async_copy(
                w2_hbm.at[te, r2, :], w2_buf.at[s, r2, :], sems.at[s, 1, k]))
        return cps

    @pl.when(b == 0)
    def _():
        for s in range(NSLOT):
            @pl.when(s < ntrans_ref[0])
            def _(s=s):
                te = trans_ref[s]
                for cp in _w_copies(te, s):
                    cp.start()

    @pl.when((wchg_ref[b] == 1) & (b < nact))
    def _():
        for cp in _w_copies(e, slot):
            cp.wait()

    @pl.when((dopref_ref[b] == 1) & (b < nact))
    def _():
        pe = prefe_ref[b]
        ns = pslot_ref[b]
        for cp in _w_copies(pe, ns):
            cp.start()

    @pl.when(b < nact)
    def _():
        xb = xs_ref[...]
        h = jnp.dot(xb, w1_buf[slot],
                    preferred_element_type=jnp.float32)
        h = jnp.maximum(h + b1_ref[0], 0.0)
        ob = jnp.dot(h, w2_buf[slot],
                     preferred_element_type=jnp.float32)
        os_ref[...] = ob + b2_ref[0]


def _ffn(bexp, nact, wslot, wchg, prefe, dopref, pslot, ntrans, trans_e,
         xs, W1, b1, W2, b2):
    grid_spec = pltpu.PrefetchScalarGridSpec(
        num_scalar_prefetch=9,
        grid=(NB,),
        in_specs=[
            pl.BlockSpec((BLK, D),
                         lambda b, be, na, *_: (jnp.minimum(b, na[0] - 1), 0)),
            pl.BlockSpec(memory_space=pl.ANY),
            pl.BlockSpec((1, 1, H), lambda b, be, *_: (be[b], 0, 0)),
            pl.BlockSpec(memory_space=pl.ANY),
            pl.BlockSpec((1, 1, D), lambda b, be, *_: (be[b], 0, 0)),
        ],
        out_specs=pl.BlockSpec((BLK, D),
                               lambda b, be, na, *_:
                               (jnp.where(b < na[0], b, NB - 1), 0)),
        scratch_shapes=[
            pltpu.VMEM((NSLOT, D, H), jnp.float32),
            pltpu.VMEM((NSLOT, H, D), jnp.float32),
            pltpu.SemaphoreType.DMA((NSLOT, 2, NSPLIT)),
        ],
    )
    return pl.pallas_call(
        _ffn_body,
        grid_spec=grid_spec,
        out_shape=jax.ShapeDtypeStruct((P, D), jnp.float32),
        compiler_params=pltpu.CompilerParams(
            dimension_semantics=("arbitrary",)),
    )(bexp, nact, wslot, wchg, prefe, dopref, pslot, ntrans, trans_e,
      xs, W1, b1, W2, b2)


def _combine_body(os_hbm, pos0_hbm, pos1_hbm, g0_hbm, g1_hbm, out_hbm,
                  idx0_v, idx1_v, g0_v, g1_v, buf0, buf1, sem0, sem1):
    wid = lax.axis_index("s") * 2 + lax.axis_index("c")
    base = wid * TW
    ld0 = pltpu.async_copy(pos0_hbm.at[pl.ds(base, TW)], idx0_v, sem0)
    ld1 = pltpu.async_copy(pos1_hbm.at[pl.ds(base, TW)], idx1_v, sem1)
    ld0.wait()
    ld1.wait()
    cp0 = pltpu.async_copy(os_hbm.at[idx0_v], buf0, sem0)
    cp1 = pltpu.async_copy(os_hbm.at[idx1_v], buf1, sem1)
    pltpu.sync_copy(g0_hbm.at[pl.ds(base, TW), :], g0_v)
    pltpu.sync_copy(g1_hbm.at[pl.ds(base, TW), :], g1_v)
    cp0.wait()
    cp1.wait()

    def row(r, _):
        gv0 = g0_v[r]
        gv1 = g1_v[r]
        for dq in range(D // 16):
            sl = pl.ds(dq * 16, 16)
            buf0[r, sl] = gv0 * buf0[r, sl] + gv1 * buf1[r, sl]
        return _

    lax.fori_loop(0, TW, row, 0)
    pltpu.sync_copy(buf0, out_hbm.at[pl.ds(base, TW), :])


def _combine(os_, pos0, pos1, g0r, g1r):
    mesh = plsc.VectorSubcoreMesh(core_axis_name="c", subcore_axis_name="s")
    return pl.kernel(
        _combine_body,
        out_type=jax.ShapeDtypeStruct((T, D), jnp.float32),
        mesh=mesh,
        scratch_types=[
            pltpu.VMEM((TW,), jnp.int32),
            pltpu.VMEM((TW,), jnp.int32),
            pltpu.VMEM((TW, 16), jnp.float32),
            pltpu.VMEM((TW, 16), jnp.float32),
            pltpu.VMEM((TW, D), jnp.float32),
            pltpu.VMEM((TW, D), jnp.float32),
            pltpu.SemaphoreType.DMA,
            pltpu.SemaphoreType.DMA,
        ],
    )(os_, pos0, pos1, g0r, g1r)


def kernel(x, Wr, br, W1, b1, W2, b2):
    B, S, Dm = x.shape
    x2 = x.reshape(T, D)
    (pos0, pos1, g0r, g1r, bexp, nact, wslot, wchg, prefe, dopref,
     pslot, ntrans, trans_e) = _router_plan(x2, Wr, br.reshape(1, N_EXP))
    pos0 = pos0.reshape(T)
    pos1 = pos1.reshape(T)
    bexp = bexp.reshape(NB)
    nact = nact.reshape(1)
    xs = _dispatch(x2, pos0, pos1)
    os_ = _ffn(bexp, nact, wslot.reshape(NB), wchg.reshape(NB),
               prefe.reshape(NB), dopref.reshape(NB), pslot.reshape(NB),
               ntrans.reshape(1), trans_e.reshape(N_EXP), xs,
               W1, b1.reshape(N_EXP, 1, H), W2, b2.reshape(N_EXP, 1, D))
    out = _combine(os_, pos0, pos1, g0r, g1r)
    return out.reshape(B, S, Dm)
